# Initial kernel scaffold; baseline (speedup 1.0000x reference)
import jax
import jax.numpy as jnp
from jax.experimental import pallas as pl


def kernel(x, embed_table, W):
    raise NotImplementedError("write your pallas kernel here")


# speedup vs baseline: 8.4781x; 8.4781x over previous
import functools

import jax
import jax.numpy as jnp
from jax import lax
from jax.experimental import pallas as pl
from jax.experimental.pallas import tpu as pltpu
from jax.experimental.pallas import tpu_sc as plsc

_VOCAB_BLOCK = 2000
_NC, _NS = 2, 16
_NW = _NC * _NS
_CHUNK = 128


def _proj_body(e_ref, w_ref, o_ref):
    o_ref[...] = lax.dot_general(
        e_ref[...], w_ref[...],
        dimension_numbers=(((1,), (1,)), ((), ())),
        preferred_element_type=jnp.float32)


def _project_table(embed_table, W):
    V, D_in = embed_table.shape
    D_out = W.shape[0]
    return pl.pallas_call(
        _proj_body,
        grid=(V // _VOCAB_BLOCK,),
        in_specs=[
            pl.BlockSpec((_VOCAB_BLOCK, D_in), lambda i: (i, 0)),
            pl.BlockSpec((D_out, D_in), lambda i: (0, 0)),
        ],
        out_specs=pl.BlockSpec((_VOCAB_BLOCK, D_out), lambda i: (i, 0)),
        out_shape=jax.ShapeDtypeStruct((V, D_out), jnp.float32),
    )(embed_table, W)


def _gather_rows(table, idx_flat):
    B = idx_flat.shape[0]
    D_pad = table.shape[1]
    b_per_w = B // _NW
    n_chunks = b_per_w // _CHUNK
    mesh = plsc.VectorSubcoreMesh(core_axis_name="c", subcore_axis_name="s")

    @functools.partial(
        pl.kernel,
        mesh=mesh,
        out_type=jax.ShapeDtypeStruct((B, D_pad), jnp.float32),
        scratch_types=[
            pltpu.VMEM((_CHUNK,), jnp.int32),
            pltpu.VMEM((_CHUNK, D_pad), jnp.float32),
            pltpu.SemaphoreType.DMA,
        ],
    )
    def k(table_hbm, idx_hbm, out_hbm, idx_v, rows_v, sem):
        wid = lax.axis_index("s") * _NC + lax.axis_index("c")
        base = wid * b_per_w

        def body(c, carry):
            off = base + c * _CHUNK
            pltpu.sync_copy(idx_hbm.at[pl.ds(off, _CHUNK)], idx_v)
            pltpu.async_copy(table_hbm.at[idx_v], rows_v, sem).wait()
            pltpu.sync_copy(rows_v, out_hbm.at[pl.ds(off, _CHUNK)])
            return carry

        lax.fori_loop(0, n_chunks, body, 0)

    return k(table, idx_flat)


def kernel(x, embed_table, W):
    B, L = x.shape
    D_out = W.shape[0]
    d_pad = (-D_out) % 128
    W_pad = jnp.pad(W, ((0, d_pad), (0, 0)))
    proj = _project_table(embed_table, W_pad)
    out_pad = _gather_rows(proj, x.reshape(-1).astype(jnp.int32))
    return out_pad[:, :D_out].reshape(B, L, D_out)


# speedup vs baseline: 9.4070x; 1.1096x over previous
import functools

import jax
import jax.numpy as jnp
from jax import lax
from jax.experimental import pallas as pl
from jax.experimental.pallas import tpu as pltpu
from jax.experimental.pallas import tpu_sc as plsc

_VOCAB_BLOCK = 2000
_NC, _NS = 2, 16
_NW = _NC * _NS
_CHUNK = 128


def _proj_body(e_ref, w_ref, o_ref):
    o_ref[...] = lax.dot_general(
        e_ref[...], w_ref[...],
        dimension_numbers=(((1,), (1,)), ((), ())),
        preferred_element_type=jnp.float32)


def _project_table(embed_table, W):
    V, D_in = embed_table.shape
    D_out = W.shape[0]
    return pl.pallas_call(
        _proj_body,
        grid=(V // _VOCAB_BLOCK,),
        in_specs=[
            pl.BlockSpec((_VOCAB_BLOCK, D_in), lambda i: (i, 0)),
            pl.BlockSpec((D_out, D_in), lambda i: (0, 0)),
        ],
        out_specs=pl.BlockSpec((_VOCAB_BLOCK, D_out), lambda i: (i, 0)),
        out_shape=jax.ShapeDtypeStruct((V, D_out), jnp.float32),
    )(embed_table, W)


def _gather_rows(table, idx_flat):
    B = idx_flat.shape[0]
    D_pad = table.shape[1]
    b_per_w = B // _NW
    n_chunks = b_per_w // _CHUNK
    mesh = plsc.VectorSubcoreMesh(core_axis_name="c", subcore_axis_name="s")

    @functools.partial(
        pl.kernel,
        mesh=mesh,
        out_type=jax.ShapeDtypeStruct((B, D_pad), jnp.float32),
        scratch_types=[
            pltpu.VMEM((b_per_w,), jnp.int32),
            pltpu.VMEM((_CHUNK, D_pad), jnp.float32),
            pltpu.VMEM((_CHUNK, D_pad), jnp.float32),
            pltpu.SemaphoreType.DMA,
            pltpu.SemaphoreType.DMA,
        ],
    )
    def k(table_hbm, idx_hbm, out_hbm, idx_v, rows0, rows1, sem0, sem1):
        wid = lax.axis_index("s") * _NC + lax.axis_index("c")
        base = wid * b_per_w
        bufs = ((rows0, sem0), (rows1, sem1))

        pltpu.sync_copy(idx_hbm.at[pl.ds(base, b_per_w)], idx_v)

        def idx_at(c):
            return idx_v.at[pl.ds(c * _CHUNK, _CHUNK)]

        pltpu.async_copy(table_hbm.at[idx_at(0)], rows0, sem0)

        def pair_body(p, carry):
            for b in (0, 1):
                cur, csem = bufs[b]
                nxt, nsem = bufs[1 - b]
                c = 2 * p + b

                @pl.when(c + 1 < n_chunks)
                def _():
                    pltpu.async_copy(table_hbm.at[idx_at(c + 1)], nxt, nsem)

                pltpu.make_async_copy(table_hbm.at[idx_at(c)], cur,
                                      csem).wait()
                pltpu.sync_copy(cur, out_hbm.at[pl.ds(base + c * _CHUNK,
                                                      _CHUNK)])
            return carry

        lax.fori_loop(0, n_chunks // 2, pair_body, 0)

    return k(table, idx_flat)


def kernel(x, embed_table, W):
    B, L = x.shape
    D_out = W.shape[0]
    d_pad = (-D_out) % 128
    W_pad = jnp.pad(W, ((0, d_pad), (0, 0)))
    proj = _project_table(embed_table, W_pad)
    out_pad = _gather_rows(proj, x.reshape(-1).astype(jnp.int32))
    return out_pad[:, :D_out].reshape(B, L, D_out)
